# baseline (speedup 1.0000x reference)
import jax
import jax.numpy as jnp
from jax import lax
from jax.experimental import pallas as pl
from jax.experimental.pallas import tpu as pltpu
from jax.experimental.pallas import tpu_sc as plsc

D_MODEL = 128
NUM_ROWS = 24
GROUP = 128
CHUNK = 256
SUPER = 4096


def _make_lookup(B: int):
    info = plsc.get_sparse_core_info()
    NC, NS = info.num_cores, info.num_subcores
    NW = NC * NS
    rows_per_w = B // NW
    assert B == NW * rows_per_w and rows_per_w % SUPER == 0
    n_pairs = rows_per_w // (2 * CHUNK)
    pairs_per_super = SUPER // (2 * CHUNK)
    chunks_per_super = SUPER // CHUNK

    mesh = plsc.VectorSubcoreMesh(core_axis_name="c", subcore_axis_name="s")

    @jax.jit
    def lookup(table, idx):
        def body(table_hbm, idx_hbm, out_hbm, table_sh, idx_s, rows0, rows1,
                 gsem0, gsem1, ssem0, ssem1):
            sid = lax.axis_index("s")
            wid = sid * NC + lax.axis_index("c")
            base = wid * rows_per_w
            rows = (rows0, rows1)
            gsems = (gsem0, gsem1)
            ssems = (ssem0, ssem1)

            @pl.when(sid == 0)
            def _():
                pltpu.sync_copy(table_hbm, table_sh)

            plsc.subcore_barrier()

            def fire_gathers(c, buf, sem):
                ioff = (c % chunks_per_super) * CHUNK
                h0 = pltpu.async_copy(
                    table_sh.at[idx_s.at[pl.ds(ioff, GROUP)]],
                    buf.at[pl.ds(0, GROUP)], sem)
                h1 = pltpu.async_copy(
                    table_sh.at[idx_s.at[pl.ds(ioff + GROUP, GROUP)]],
                    buf.at[pl.ds(GROUP, GROUP)], sem)
                return h0, h1

            def fire_store(c, buf, sem):
                pltpu.async_copy(
                    buf, out_hbm.at[pl.ds(base + c * CHUNK, CHUNK)], sem)

            def drain(buf, sem):
                pltpu.make_async_copy(
                    buf, out_hbm.at[pl.ds(0, CHUNK)], sem).wait()

            def pair(p, carry):
                @pl.when(p >= 1)
                def _():
                    drain(rows[1], gsems[1])
                    fire_store(2 * p - 1, rows[1], ssems[1])

                @pl.when(p % pairs_per_super == 0)
                def _():
                    pltpu.sync_copy(
                        idx_hbm.at[pl.ds(base + (p // pairs_per_super) * SUPER,
                                         SUPER)],
                        idx_s)

                @pl.when(p >= 1)
                def _():
                    drain(rows[0], ssems[0])

                g0, g1 = fire_gathers(2 * p, rows[0], gsems[0])

                @pl.when(p >= 1)
                def _():
                    drain(rows[1], ssems[1])

                fire_gathers(2 * p + 1, rows[1], gsems[1])

                g0.wait()
                g1.wait()
                fire_store(2 * p, rows[0], ssems[0])
                return carry

            lax.fori_loop(0, n_pairs, pair, 0, unroll=False)

            drain(rows[1], gsems[1])
            fire_store(2 * n_pairs - 1, rows[1], ssems[1])
            drain(rows[0], ssems[0])
            drain(rows[1], ssems[1])

        return pl.kernel(
            body,
            out_type=jax.ShapeDtypeStruct((B, D_MODEL), jnp.float32),
            mesh=mesh,
            scratch_types=[
                pltpu.VMEM_SHARED((NUM_ROWS, D_MODEL), jnp.float32),
                pltpu.VMEM((SUPER,), jnp.int32),
                pltpu.VMEM((CHUNK, D_MODEL), jnp.float32),
                pltpu.VMEM((CHUNK, D_MODEL), jnp.float32),
                pltpu.SemaphoreType.DMA,
                pltpu.SemaphoreType.DMA,
                pltpu.SemaphoreType.DMA,
                pltpu.SemaphoreType.DMA,
            ],
        )(table, idx)

    return lookup


def kernel(hours, hour_emb):
    B = hours.size
    flat = hours.reshape(B).astype(jnp.int32)
    out = _make_lookup(B)(hour_emb, flat)
    return out.reshape(*hours.shape, D_MODEL)


# speedup vs baseline: 1.4811x; 1.4811x over previous
import jax
import jax.numpy as jnp
from jax import lax
from jax.experimental import pallas as pl
from jax.experimental.pallas import tpu as pltpu
from jax.experimental.pallas import tpu_sc as plsc

D_MODEL = 128
NUM_ROWS = 24
GROUP = 128
CHUNK = 256
SUPER = 2048


def _make_lookup(B: int):
    info = plsc.get_sparse_core_info()
    NC, NS = info.num_cores, info.num_subcores
    NW = NC * NS
    rows_per_w = B // NW
    assert B == NW * rows_per_w and rows_per_w % (2 * SUPER) == 0
    pairs_per_super = SUPER // (2 * CHUNK)
    n_super_pairs = rows_per_w // (2 * SUPER)

    mesh = plsc.VectorSubcoreMesh(core_axis_name="c", subcore_axis_name="s")

    @jax.jit
    def lookup(table, idx):
        def body(table_hbm, idx_hbm, out_hbm, table_sh, idx0, idx1,
                 rows0, rows1, gsem0, gsem1, ssem0, ssem1, isem0, isem1):
            sid = lax.axis_index("s")
            wid = sid * NC + lax.axis_index("c")
            base = wid * rows_per_w
            rows = (rows0, rows1)
            idxs = (idx0, idx1)
            gsems = (gsem0, gsem1)
            ssems = (ssem0, ssem1)
            isems = (isem0, isem1)

            @pl.when(sid == 0)
            def _():
                pltpu.sync_copy(table_hbm, table_sh)

            plsc.subcore_barrier()

            def fire_idx_load(s, ib):
                pltpu.async_copy(idx_hbm.at[pl.ds(base + s * SUPER, SUPER)],
                                 idxs[ib], isems[ib])

            def wait_idx(ib):
                pltpu.make_async_copy(idx_hbm.at[pl.ds(0, SUPER)], idxs[ib],
                                      isems[ib]).wait()

            def process_super(sbase, idx_ref, wait_cond):
                def pq(q, carry):
                    for b in (0, 1):
                        ioff = (2 * q + b) * CHUNK

                        @pl.when(wait_cond(q))
                        def _():
                            pltpu.make_async_copy(
                                rows[b], out_hbm.at[pl.ds(0, CHUNK)],
                                ssems[b]).wait()

                        g0 = pltpu.async_copy(
                            table_sh.at[idx_ref.at[pl.ds(ioff, GROUP)]],
                            rows[b].at[pl.ds(0, GROUP)], gsems[b])
                        g1 = pltpu.async_copy(
                            table_sh.at[idx_ref.at[pl.ds(ioff + GROUP,
                                                         GROUP)]],
                            rows[b].at[pl.ds(GROUP, GROUP)], gsems[b])
                        g0.wait()
                        g1.wait()
                        pltpu.async_copy(
                            rows[b], out_hbm.at[pl.ds(sbase + ioff, CHUNK)],
                            ssems[b])
                    return carry

                lax.fori_loop(0, pairs_per_super, pq, 0, unroll=False)

            fire_idx_load(0, 0)
            fire_idx_load(1, 1)

            def super_pair(sp, carry):
                wait_idx(0)
                process_super(base + (2 * sp) * SUPER, idxs[0],
                              lambda q: (sp > 0) | (q > 0))

                @pl.when(sp < n_super_pairs - 1)
                def _():
                    fire_idx_load(2 * sp + 2, 0)

                wait_idx(1)
                process_super(base + (2 * sp + 1) * SUPER, idxs[1],
                              lambda q: q >= 0)

                @pl.when(sp < n_super_pairs - 1)
                def _():
                    fire_idx_load(2 * sp + 3, 1)
                return carry

            lax.fori_loop(0, n_super_pairs, super_pair, 0, unroll=False)

            for b in (0, 1):
                pltpu.make_async_copy(
                    rows[b], out_hbm.at[pl.ds(0, CHUNK)], ssems[b]).wait()

        return pl.kernel(
            body,
            out_type=jax.ShapeDtypeStruct((B, D_MODEL), jnp.float32),
            mesh=mesh,
            scratch_types=[
                pltpu.VMEM_SHARED((NUM_ROWS, D_MODEL), jnp.float32),
                pltpu.VMEM((SUPER,), jnp.int32),
                pltpu.VMEM((SUPER,), jnp.int32),
                pltpu.VMEM((CHUNK, D_MODEL), jnp.float32),
                pltpu.VMEM((CHUNK, D_MODEL), jnp.float32),
                pltpu.SemaphoreType.DMA,
                pltpu.SemaphoreType.DMA,
                pltpu.SemaphoreType.DMA,
                pltpu.SemaphoreType.DMA,
                pltpu.SemaphoreType.DMA,
                pltpu.SemaphoreType.DMA,
            ],
        )(table, idx)

    return lookup


def kernel(hours, hour_emb):
    B = hours.size
    flat = hours.reshape(B).astype(jnp.int32)
    out = _make_lookup(B)(hour_emb, flat)
    return out.reshape(*hours.shape, D_MODEL)
